# Initial kernel scaffold; baseline (speedup 1.0000x reference)
import jax
import jax.numpy as jnp
from jax.experimental import pallas as pl


def kernel(x, edge_index1, edge_index2, lin1_W, lin1_b, lin2_W, lin2_b, beta1, beta2):
    raise NotImplementedError("write your pallas kernel here")


# speedup vs baseline: 25.5101x; 25.5101x over previous
import functools

import jax
import jax.numpy as jnp
from jax import lax
from jax.experimental import pallas as pl
from jax.experimental.pallas import tpu as pltpu
from jax.experimental.pallas import tpu_sc as plsc

_N = 100000
_E = 1600000
_D_IN = 128
_HID = 16
_N_CLS = 41

_NC = 2
_NS = 16
_NW = _NC * _NS

_BLK = 256
_KSUB = 2
_NB = 196
_EPW = _BLK * _NB
_E_PAD = _NW * _EPW
_RPT = 6272
_N_ACC = _NS * _RPT
_ZCH = 128


def _rsqrt16(z):
    i = plsc.bitcast(z, jnp.int32)
    y = plsc.bitcast(jnp.full((16,), 0x5F3759DF, jnp.int32) - (i >> 1), jnp.float32)
    half = jnp.full((16,), 0.5, jnp.float32)
    threehalf = jnp.full((16,), 1.5, jnp.float32)
    hz = half * z
    for _ in range(3):
        y = y * (threehalf - hz * y * y)
    return y


def _agnn_edge_pass(h_pad, src2, dst2, beta_vec):
    mesh = plsc.VectorSubcoreMesh(core_axis_name="c", subcore_axis_name="s")

    @functools.partial(
        pl.kernel,
        mesh=mesh,
        compiler_params=pltpu.CompilerParams(needs_layout_passes=False,
                                             use_tc_tiling_on_sc=False),
        out_type=(
            jax.ShapeDtypeStruct((_NC, _N_ACC, _HID), jnp.float32),
            jax.ShapeDtypeStruct((_NC, _N_ACC), jnp.float32),
        ),
        scratch_types=[
            pltpu.VMEM_SHARED((_N_ACC, _HID), jnp.float32),
            pltpu.VMEM_SHARED((_N_ACC,), jnp.float32),
            pltpu.VMEM((_KSUB, 128), jnp.int32),
            pltpu.VMEM((_KSUB, 128), jnp.int32),
            pltpu.VMEM((_BLK, _HID), jnp.float32),
            pltpu.VMEM((_BLK, _HID), jnp.float32),
            pltpu.VMEM((_BLK, _HID), jnp.float32),
            pltpu.VMEM((_BLK,), jnp.float32),
            pltpu.VMEM((_ZCH, _HID), jnp.float32),
            pltpu.VMEM((_ZCH,), jnp.float32),
            pltpu.VMEM((16,), jnp.float32),
            pltpu.SemaphoreType.DMA,
        ],
    )
    def run(h_hbm, src_hbm, dst_hbm, beta_hbm, out_hbm, s_hbm,
            acc, sacc, sidx, didx, srows, drows, msg, exb, zbuf, zbuf1,
            betav, sem):
        cid = lax.axis_index("c")
        sid = lax.axis_index("s")
        wid = cid * _NS + sid

        zv = jnp.zeros((16,), jnp.float32)

        def zrow(i, c):
            zbuf[i] = zv
            return c

        lax.fori_loop(0, _ZCH, zrow, 0)

        def zrow1(i, c):
            zbuf1[pl.ds(i * 16, 16)] = zv
            return c

        lax.fori_loop(0, _ZCH // 16, zrow1, 0)

        row_t = sid * _RPT
        for k in range(_RPT // _ZCH):
            pltpu.sync_copy(zbuf, acc.at[pl.ds(row_t + k * _ZCH, _ZCH)])
            pltpu.sync_copy(zbuf1, sacc.at[pl.ds(row_t + k * _ZCH, _ZCH)])

        pltpu.sync_copy(beta_hbm, betav)
        plsc.subcore_barrier()

        bvec = betav[...]
        eps2 = jnp.full((16,), 1e-24, jnp.float32)
        lane = lax.iota(jnp.int32, 16)
        r0 = wid * (_NB * _KSUB)

        def block_body(b, carry):
            rb = r0 + b * _KSUB
            pltpu.sync_copy(src_hbm.at[pl.ds(rb, _KSUB)], sidx)
            pltpu.sync_copy(dst_hbm.at[pl.ds(rb, _KSUB)], didx)
            cps = []
            for j in range(_KSUB):
                cps.append(pltpu.async_copy(
                    h_hbm.at[sidx.at[j]], srows.at[pl.ds(j * 128, 128)], sem))
                cps.append(pltpu.async_copy(
                    h_hbm.at[didx.at[j]], drows.at[pl.ds(j * 128, 128)], sem))
            for cp in cps:
                cp.wait()

            def group(gg, c2):
                row = gg * 16 + lane
                ss = zv
                dd = zv
                sd = zv
                avals = []
                for f in range(_HID):
                    col = jnp.full((16,), f, jnp.int32)
                    a = plsc.load_gather(srows, [row, col])
                    bb = plsc.load_gather(drows, [row, col])
                    avals.append(a)
                    ss = ss + a * a
                    dd = dd + bb * bb
                    sd = sd + a * bb
                inv = _rsqrt16(jnp.maximum(ss, eps2)) * _rsqrt16(jnp.maximum(dd, eps2))
                ex = jnp.exp(bvec * sd * inv)
                exb[pl.ds(gg * 16, 16)] = ex
                for f in range(_HID):
                    col = jnp.full((16,), f, jnp.int32)
                    plsc.store_scatter(msg, [row, col], avals[f] * ex)
                return c2

            lax.fori_loop(0, _BLK // 16, group, carry)

            for j in range(_KSUB):
                pltpu.sync_copy(msg.at[pl.ds(j * 128, 128)],
                                acc.at[didx.at[j]], add=True)
                pltpu.sync_copy(exb.at[pl.ds(j * 128, 128)],
                                sacc.at[didx.at[j]], add=True)
            return carry

        lax.fori_loop(0, _NB, block_body, 0)

        plsc.subcore_barrier()
        pltpu.sync_copy(acc.at[pl.ds(row_t, _RPT)],
                        out_hbm.at[cid, pl.ds(row_t, _RPT)])
        pltpu.sync_copy(sacc.at[pl.ds(row_t, _RPT)],
                        s_hbm.at[cid, pl.ds(row_t, _RPT)])

    return run(h_pad, src2, dst2, beta_vec)


def _lin1_kernel(x_ref, w_ref, b_ref, o_ref):
    h = lax.dot_general(x_ref[...], w_ref[...], (((1,), (1,)), ((), ())),
                        preferred_element_type=jnp.float32)
    o_ref[...] = jnp.maximum(h + b_ref[...], 0.0)


def _lin1(x, w1, b1):
    rb = 800
    grid = (_N // rb,)
    return pl.pallas_call(
        _lin1_kernel,
        grid=grid,
        in_specs=[
            pl.BlockSpec((rb, _D_IN), lambda i: (i, 0)),
            pl.BlockSpec((_HID, _D_IN), lambda i: (0, 0)),
            pl.BlockSpec((1, _HID), lambda i: (0, 0)),
        ],
        out_specs=pl.BlockSpec((rb, _HID), lambda i: (i, 0)),
        out_shape=jax.ShapeDtypeStruct((_N, _HID), jnp.float32),
    )(x, w1, b1.reshape(1, _HID))


def _epi_kernel(p0_ref, p1_ref, s0_ref, s1_ref, h_ref, beta_ref, o_ref):
    h = h_ref[...]
    n2 = jnp.sum(h * h, axis=1, keepdims=True)
    nrm = jnp.maximum(jnp.sqrt(n2), 1e-12)
    selfex = jnp.exp(beta_ref[0, 0] * n2 / (nrm * nrm))
    stot = s0_ref[...] + s1_ref[...] + selfex + 1e-16
    o_ref[...] = (p0_ref[...] + p1_ref[...] + selfex * h) / stot


def _epilogue(parts, h_pad, beta):
    op, sp = parts
    p0, p1 = op[0], op[1]
    s0 = sp[0].reshape(_N_ACC, 1)
    s1 = sp[1].reshape(_N_ACC, 1)
    rb = _ZCH
    grid = (_N_ACC // rb,)
    return pl.pallas_call(
        _epi_kernel,
        grid=grid,
        in_specs=[
            pl.BlockSpec((rb, _HID), lambda i: (i, 0)),
            pl.BlockSpec((rb, _HID), lambda i: (i, 0)),
            pl.BlockSpec((rb, 1), lambda i: (i, 0)),
            pl.BlockSpec((rb, 1), lambda i: (i, 0)),
            pl.BlockSpec((rb, _HID), lambda i: (i, 0)),
            pl.BlockSpec((1, 1), lambda i: (0, 0)),
        ],
        out_specs=pl.BlockSpec((rb, _HID), lambda i: (i, 0)),
        out_shape=jax.ShapeDtypeStruct((_N_ACC, _HID), jnp.float32),
    )(p0, p1, s0, s1, h_pad, beta.reshape(1, 1))


def _final_kernel(p0_ref, p1_ref, s0_ref, s1_ref, h_ref, beta_ref,
                  w_ref, b_ref, o_ref):
    h = h_ref[...]
    n2 = jnp.sum(h * h, axis=1, keepdims=True)
    nrm = jnp.maximum(jnp.sqrt(n2), 1e-12)
    selfex = jnp.exp(beta_ref[0, 0] * n2 / (nrm * nrm))
    stot = s0_ref[...] + s1_ref[...] + selfex + 1e-16
    h3 = (p0_ref[...] + p1_ref[...] + selfex * h) / stot
    l = lax.dot_general(h3, w_ref[...], (((1,), (1,)), ((), ())),
                        preferred_element_type=jnp.float32) + b_ref[...]
    m = jnp.max(l, axis=1, keepdims=True)
    e = jnp.exp(l - m)
    o_ref[...] = (l - m) - jnp.log(jnp.sum(e, axis=1, keepdims=True))


def _final(parts, h_pad, beta, w2, b2):
    op, sp = parts
    p0, p1 = op[0], op[1]
    s0 = sp[0].reshape(_N_ACC, 1)
    s1 = sp[1].reshape(_N_ACC, 1)
    rb = 800
    grid = (_N // rb,)
    return pl.pallas_call(
        _final_kernel,
        grid=grid,
        in_specs=[
            pl.BlockSpec((rb, _HID), lambda i: (i, 0)),
            pl.BlockSpec((rb, _HID), lambda i: (i, 0)),
            pl.BlockSpec((rb, 1), lambda i: (i, 0)),
            pl.BlockSpec((rb, 1), lambda i: (i, 0)),
            pl.BlockSpec((rb, _HID), lambda i: (i, 0)),
            pl.BlockSpec((1, 1), lambda i: (0, 0)),
            pl.BlockSpec((_N_CLS, _HID), lambda i: (0, 0)),
            pl.BlockSpec((1, _N_CLS), lambda i: (0, 0)),
        ],
        out_specs=pl.BlockSpec((rb, _N_CLS), lambda i: (i, 0)),
        out_shape=jax.ShapeDtypeStruct((_N, _N_CLS), jnp.float32),
    )(p0, p1, s0, s1, h_pad, beta.reshape(1, 1), w2, b2.reshape(1, _N_CLS))


def _pad_edges(edge_index):
    src = edge_index[0]
    dst = edge_index[1]
    padn = _E_PAD - _E
    fill = jnp.full((padn,), _N, jnp.int32)
    src2 = jnp.concatenate([src, fill]).reshape(_E_PAD // 128, 128)
    dst2 = jnp.concatenate([dst, fill]).reshape(_E_PAD // 128, 128)
    return src2, dst2


def kernel(x, edge_index1, edge_index2, lin1_W, lin1_b, lin2_W, lin2_b,
           beta1, beta2):
    h1 = _lin1(x, lin1_W, lin1_b)
    h1p = jnp.pad(h1, ((0, _N_ACC - _N), (0, 0)))
    s1_, d1_ = _pad_edges(edge_index1)
    s2_, d2_ = _pad_edges(edge_index2)
    bvec1 = jnp.broadcast_to(beta1, (16,)).astype(jnp.float32)
    bvec2 = jnp.broadcast_to(beta2, (16,)).astype(jnp.float32)

    parts1 = _agnn_edge_pass(h1p, s1_, d1_, bvec1)
    h2p = _epilogue(parts1, h1p, beta1)
    parts2 = _agnn_edge_pass(h2p, s2_, d2_, bvec2)
    return _final(parts2, h2p, beta2, lin2_W, lin2_b)
